# baseline (speedup 1.0000x reference)
import functools

import jax
import jax.numpy as jnp
from jax import lax
from jax.experimental import pallas as pl
from jax.experimental.pallas import tpu as pltpu
from jax.experimental.pallas import tpu_sc as plsc

D = 32
NC = 2
NS = 16
NW = NC * NS
CHUNK = 80
NBUF = 4


def _emb_body(idx_hbm, table_hbm, out_hbm, idx_v, *bufs_and_sems):
    bufs = bufs_and_sems[:NBUF]
    gsems = bufs_and_sems[NBUF:2 * NBUF]
    ssems = bufs_and_sems[2 * NBUF:3 * NBUF]
    n_rows = out_hbm.shape[0]
    n_cols = out_hbm.shape[1]
    b_per_w = (n_rows * n_cols) // NW
    n_chunk = b_per_w // CHUNK
    rows_per_chunk = CHUNK // n_cols

    wid = lax.axis_index("s") * NC + lax.axis_index("c")
    base = wid * b_per_w
    pltpu.sync_copy(idx_hbm.at[pl.ds(base, b_per_w)], idx_v)
    row_base = base // n_cols

    def store(j, buf, sem):
        r0 = row_base + j * rows_per_chunk
        for r in range(rows_per_chunk):
            pltpu.async_copy(buf.at[pl.ds(r * n_cols, n_cols)],
                             out_hbm.at[r0 + r], sem)

    def wait_store(j, buf, sem):
        r0 = row_base + j * rows_per_chunk
        for r in range(rows_per_chunk):
            pltpu.make_async_copy(buf.at[pl.ds(r * n_cols, n_cols)],
                                  out_hbm.at[r0 + r], sem).wait()

    for b in range(NBUF):
        pltpu.async_copy(table_hbm.at[idx_v.at[pl.ds(b * CHUNK, CHUNK)]],
                         bufs[b], gsems[b])

    def group(g, carry):
        for b in range(NBUF):
            j = g * NBUF + b
            pltpu.make_async_copy(table_hbm.at[idx_v.at[pl.ds(0, CHUNK)]],
                                  bufs[b], gsems[b]).wait()
            store(j, bufs[b], ssems[b])

            @pl.when(j + NBUF < n_chunk)
            def _():
                wait_store(j, bufs[b], ssems[b])
                pltpu.async_copy(
                    table_hbm.at[idx_v.at[pl.ds((j + NBUF) * CHUNK, CHUNK)]],
                    bufs[b], gsems[b])
        return carry

    lax.fori_loop(0, n_chunk // NBUF, group, 0)

    for b in range(NBUF):
        wait_store(n_chunk - NBUF + b, bufs[b], ssems[b])


def kernel(ids, embeddings):
    n0, n1 = ids.shape
    B = n0 * n1
    b_per_w = B // NW
    idx = ids.reshape(B).astype(jnp.int32)
    mesh = plsc.VectorSubcoreMesh(core_axis_name="c", subcore_axis_name="s")
    out = pl.kernel(
        _emb_body,
        out_type=jax.ShapeDtypeStruct((n0, n1, D), jnp.float32),
        mesh=mesh,
        scratch_types=(
            [pltpu.VMEM((b_per_w,), jnp.int32)]
            + [pltpu.VMEM((CHUNK, D), jnp.float32)] * NBUF
            + [pltpu.SemaphoreType.DMA] * (2 * NBUF)
        ),
        compiler_params=pltpu.CompilerParams(use_tc_tiling_on_sc=False),
    )(idx, embeddings)
    return out
